# Initial kernel scaffold; baseline (speedup 1.0000x reference)
import jax
import jax.numpy as jnp
from jax.experimental import pallas as pl


def kernel(adv_logits, A):
    raise NotImplementedError("write your pallas kernel here")


# speedup vs baseline: 2.9294x; 2.9294x over previous
import functools
import jax
import jax.numpy as jnp
from jax.experimental import pallas as pl
from jax.experimental.pallas import tpu as pltpu


def _body(x_ref, a_ref, out_ref, accq_ref, accm_ref, *, nsteps):
    step = pl.program_id(0)

    @pl.when(step == 0)
    def _init():
        accq_ref[...] = jnp.zeros_like(accq_ref)
        accm_ref[...] = jnp.zeros_like(accm_ref)

    x = x_ref[...]
    a = a_ref[...]
    c = jnp.max(x)
    e = jnp.exp(x - c)

    li = jax.lax.broadcasted_iota(jnp.int32, (128, 128), 0)
    mi = jax.lax.broadcasted_iota(jnp.int32, (128, 128), 1)
    g8 = ((li // 8) == (mi // 8)).astype(jnp.float32)
    denom = jax.lax.dot(e, g8, preferred_element_type=jnp.float32)
    denom = jnp.maximum(denom, 1e-30)

    gi = jax.lax.broadcasted_iota(jnp.int32, (16, 128), 0)
    l2 = jax.lax.broadcasted_iota(jnp.int32, (16, 128), 1)
    em = (gi == (l2 // 8)).astype(jnp.float32)
    aexp = jax.lax.dot(a.astype(jnp.float32), em,
                       preferred_element_type=jnp.float32)
    lane_mod = (jax.lax.broadcasted_iota(jnp.int32, x.shape, 1) % 8
                ).astype(jnp.float32)
    mask = lane_mod == aexp
    q = jnp.where(mask, e, 0.0) / denom
    accq_ref[...] += jnp.sum(q, axis=0, keepdims=True)
    accm_ref[...] += jnp.sum(mask.astype(jnp.float32), axis=0, keepdims=True)

    @pl.when(step == nsteps - 1)
    def _fin():
        fl = jax.lax.broadcasted_iota(jnp.int32, (128, 8), 0)
        fk = jax.lax.broadcasted_iota(jnp.int32, (128, 8), 1)
        p8 = ((fl % 8) == fk).astype(jnp.float32)
        s8 = jax.lax.dot(accq_ref[...], p8, preferred_element_type=jnp.float32,
                         precision=jax.lax.Precision.HIGHEST)
        c8 = jax.lax.dot(accm_ref[...], p8, preferred_element_type=jnp.float32,
                         precision=jax.lax.Precision.HIGHEST)
        term = jnp.where(c8 > 0, 2.0 * c8 - 2.0 * s8, 0.0) / jnp.where(
            c8 > 0, c8, 1.0)
        out_ref[...] = jnp.sum(term, axis=(0, 1), keepdims=True) - 1.0


def kernel(adv_logits, A):
    n, k = adv_logits.shape
    assert k == 8
    rows = n // 16
    x2 = adv_logits.reshape(rows, 128)
    a2 = A.astype(jnp.int32).reshape(rows, 16)
    nsteps = 16
    b = rows // nsteps
    out = pl.pallas_call(
        functools.partial(_body, nsteps=nsteps),
        grid=(nsteps,),
        in_specs=[
            pl.BlockSpec((b, 128), lambda i: (i, 0)),
            pl.BlockSpec((b, 16), lambda i: (i, 0)),
        ],
        out_specs=pl.BlockSpec((1, 1), lambda i: (0, 0)),
        out_shape=jax.ShapeDtypeStruct((1, 1), jnp.float32),
        scratch_shapes=[
            pltpu.VMEM((1, 128), jnp.float32),
            pltpu.VMEM((1, 128), jnp.float32),
        ],
    )(x2, a2)
    return out[0, 0]
